# baseline (speedup 1.0000x reference)
import jax
import jax.numpy as jnp
from jax.experimental import pallas as pl
from jax.experimental.pallas import tpu as pltpu

_H = 16
_HD = 64
_TOPK = 32
_RB = 256
_BIASW = 2304

_HIGH = jax.lax.Precision.HIGHEST

_SORT8 = [(0, 1), (2, 3), (4, 5), (6, 7),
          (0, 2), (1, 3), (4, 6), (5, 7),
          (1, 2), (5, 6),
          (0, 4), (1, 5), (2, 6), (3, 7),
          (2, 4), (3, 5),
          (1, 2), (3, 4), (5, 6)]


def _ln(x, g, b, eps=1e-5):
    mu = jnp.mean(x, axis=1, keepdims=True)
    xc = x - mu
    var = jnp.mean(xc * xc, axis=1, keepdims=True)
    return xc * jax.lax.rsqrt(var + eps) * g + b


def _qkv_kernel(x_ref, w_ref, b_ref, g_ref, beta_ref, o_ref):
    xn = _ln(x_ref[...], g_ref[...], beta_ref[...])
    o_ref[...] = jax.lax.dot_general(
        xn, w_ref[...], (((1,), (0,)), ((), ())),
        precision=_HIGH, preferred_element_type=jnp.float32) + b_ref[...]


def _attn_kernel(q_ref, kt_ref, v_ref, relw_ref, o_ref):
    n = kt_ref.shape[2]
    scale = _HD ** -0.5
    q = q_ref[0]
    kt = kt_ref[0]
    logits = jax.lax.dot_general(
        q, kt, (((1,), (0,)), ((), ())),
        precision=_HIGH, preferred_element_type=jnp.float32) * scale

    m = jnp.broadcast_to(relw_ref[0], (_RB, _BIASW))
    r = jax.lax.broadcasted_iota(jnp.int32, (_RB, _BIASW), 0)
    for k in range(8):
        sh = 1 << k
        mrot = jnp.roll(m, -sh, axis=1)
        m = jnp.where(((r >> k) & 1) == 0, mrot, m)
    logits = logits + m[:, :n]

    c = n // 8
    ch = [logits[:, i * c:(i + 1) * c] for i in range(8)]
    for i, j in _SORT8:
        hi = jnp.maximum(ch[i], ch[j])
        lo = jnp.minimum(ch[i], ch[j])
        ch[i], ch[j] = hi, lo

    rowmax = jnp.max(ch[0], axis=1, keepdims=True)
    neg = jnp.float32(-jnp.inf)

    def body(_, carry):
        c0, c1, c2, c3, c4, c5, c6, c7, _m = carry
        cur = jnp.max(c0, axis=1, keepdims=True)
        hit = c0 == cur
        c0 = jnp.where(hit, c1, c0)
        c1 = jnp.where(hit, c2, c1)
        c2 = jnp.where(hit, c3, c2)
        c3 = jnp.where(hit, c4, c3)
        c4 = jnp.where(hit, c5, c4)
        c5 = jnp.where(hit, c6, c5)
        c6 = jnp.where(hit, c7, c6)
        c7 = jnp.where(hit, neg, c7)
        return (c0, c1, c2, c3, c4, c5, c6, c7, cur)

    out = jax.lax.fori_loop(0, _TOPK, body, tuple(ch) + (rowmax,))
    thresh = out[8]

    p = jnp.where(logits >= thresh, jnp.exp(logits - rowmax), 0.0)
    denom = jnp.sum(p, axis=1, keepdims=True)
    pv = jax.lax.dot_general(
        p.astype(jnp.bfloat16), v_ref[0], (((1,), (0,)), ((), ())),
        preferred_element_type=jnp.float32)
    o_ref[0] = pv / denom


def _ffn_kernel(x_ref, ao_ref, wo_ref, bo_ref, g2_ref, b2_ref,
                w1_ref, bf1_ref, w2_ref, bf2_ref, o_ref):
    x = x_ref[...]
    proj = jax.lax.dot_general(
        ao_ref[...].astype(jnp.bfloat16), wo_ref[...], (((1,), (0,)), ((), ())),
        preferred_element_type=jnp.float32)
    x1 = x + proj + bo_ref[...]
    xn2 = _ln(x1, g2_ref[...], b2_ref[...])
    h = jax.lax.dot_general(
        xn2.astype(jnp.bfloat16), w1_ref[...], (((1,), (0,)), ((), ())),
        preferred_element_type=jnp.float32) + bf1_ref[...]
    h = 0.5 * h * (1.0 + jax.lax.erf(h * 0.7071067811865476))
    ff = jax.lax.dot_general(
        h.astype(jnp.bfloat16), w2_ref[...], (((1,), (0,)), ((), ())),
        preferred_element_type=jnp.float32) + bf2_ref[...]
    o_ref[...] = x1 + ff


def kernel(x, Wq, bq, Wk, bk, Wv, bv, Wo, bo, g1, beta1, g2, beta2,
           W1, bf1, W2, bf2, rel_emb):
    b, n, d = x.shape
    nb = n // _RB
    x2 = x.reshape(n, d)

    wqkv = jnp.concatenate([Wq.T, Wk.T, Wv.T], axis=1)
    bqkv = jnp.concatenate([bq, bk, bv]).reshape(1, 3 * d)
    y = pl.pallas_call(
        _qkv_kernel,
        grid=(nb,),
        in_specs=[
            pl.BlockSpec((_RB, d), lambda i: (i, 0)),
            pl.BlockSpec((d, 3 * d), lambda i: (0, 0)),
            pl.BlockSpec((1, 3 * d), lambda i: (0, 0)),
            pl.BlockSpec((1, d), lambda i: (0, 0)),
            pl.BlockSpec((1, d), lambda i: (0, 0)),
        ],
        out_specs=pl.BlockSpec((_RB, 3 * d), lambda i: (i, 0)),
        out_shape=jax.ShapeDtypeStruct((n, 3 * d), jnp.float32),
    )(x2, wqkv, bqkv, g1.reshape(1, d), beta1.reshape(1, d))

    q, kk, v = jnp.split(y, 3, axis=1)
    qh = q.reshape(n, _H, _HD).transpose(1, 0, 2)
    kth = kk.reshape(n, _H, _HD).transpose(1, 2, 0)
    vh = v.reshape(n, _H, _HD).transpose(1, 0, 2).astype(jnp.bfloat16)

    maxseq = (rel_emb.shape[0] + 1) // 2
    relt = jnp.pad(rel_emb.T, ((0, 0), (0, 1)))
    starts = [maxseq - _RB - bi * _RB for bi in range(nb)]
    relw = jnp.stack(
        [relt[:, s:s + _BIASW] for s in starts], axis=1)
    relw = relw.reshape(_H * nb, 1, _BIASW)

    ao = pl.pallas_call(
        _attn_kernel,
        grid=(_H, nb),
        in_specs=[
            pl.BlockSpec((1, _RB, _HD), lambda h, bi: (h, bi, 0)),
            pl.BlockSpec((1, _HD, n), lambda h, bi: (h, 0, 0)),
            pl.BlockSpec((1, n, _HD), lambda h, bi: (h, 0, 0)),
            pl.BlockSpec((1, 1, _BIASW), lambda h, bi, nb=nb: (h * nb + bi, 0, 0)),
        ],
        out_specs=pl.BlockSpec((1, _RB, _HD), lambda h, bi: (h, bi, 0)),
        out_shape=jax.ShapeDtypeStruct((_H, n, _HD), jnp.float32),
    )(qh, kth, vh, relw)
    ao2 = ao.transpose(1, 0, 2).reshape(n, d)

    out = pl.pallas_call(
        _ffn_kernel,
        grid=(nb,),
        in_specs=[
            pl.BlockSpec((_RB, d), lambda i: (i, 0)),
            pl.BlockSpec((_RB, d), lambda i: (i, 0)),
            pl.BlockSpec((d, d), lambda i: (0, 0)),
            pl.BlockSpec((1, d), lambda i: (0, 0)),
            pl.BlockSpec((1, d), lambda i: (0, 0)),
            pl.BlockSpec((1, d), lambda i: (0, 0)),
            pl.BlockSpec((d, 4 * d), lambda i: (0, 0)),
            pl.BlockSpec((1, 4 * d), lambda i: (0, 0)),
            pl.BlockSpec((4 * d, d), lambda i: (0, 0)),
            pl.BlockSpec((1, d), lambda i: (0, 0)),
        ],
        out_specs=pl.BlockSpec((_RB, d), lambda i: (i, 0)),
        out_shape=jax.ShapeDtypeStruct((n, d), jnp.float32),
    )(x2, ao2, Wo.T.astype(jnp.bfloat16), bo.reshape(1, d),
      g2.reshape(1, d), beta2.reshape(1, d),
      W1.T.astype(jnp.bfloat16), bf1.reshape(1, 4 * d),
      W2.T.astype(jnp.bfloat16), bf2.reshape(1, d))

    return out.reshape(b, n, d)


# speedup vs baseline: 1.3391x; 1.3391x over previous
import jax
import jax.numpy as jnp
from jax.experimental import pallas as pl
from jax.experimental.pallas import tpu as pltpu

_H = 16
_HD = 64
_TOPK = 32
_RB = 256
_BIASW = 2304

_HIGH = jax.lax.Precision.HIGHEST

_SORT8 = [(0, 1), (2, 3), (4, 5), (6, 7),
          (0, 2), (1, 3), (4, 6), (5, 7),
          (1, 2), (5, 6),
          (0, 4), (1, 5), (2, 6), (3, 7),
          (2, 4), (3, 5),
          (1, 2), (3, 4), (5, 6)]


def _ln(x, g, b, eps=1e-5):
    mu = jnp.mean(x, axis=1, keepdims=True)
    xc = x - mu
    var = jnp.mean(xc * xc, axis=1, keepdims=True)
    return xc * jax.lax.rsqrt(var + eps) * g + b


def _qkv_kernel(x_ref, w_ref, b_ref, g_ref, beta_ref, o_ref):
    xn = _ln(x_ref[...], g_ref[...], beta_ref[...])
    o_ref[...] = jax.lax.dot_general(
        xn, w_ref[...], (((1,), (0,)), ((), ())),
        precision=_HIGH, preferred_element_type=jnp.float32) + b_ref[...]


def _attn_kernel(q_ref, kt_ref, v_ref, relw_ref, o_ref):
    n = kt_ref.shape[2]
    scale = _HD ** -0.5
    q = q_ref[0]
    kt = kt_ref[0]
    logits = jax.lax.dot_general(
        q, kt, (((1,), (0,)), ((), ())),
        precision=_HIGH, preferred_element_type=jnp.float32) * scale

    m = jnp.broadcast_to(relw_ref[0], (_RB, _BIASW))
    r = jax.lax.broadcasted_iota(jnp.int32, (_RB, _BIASW), 0)
    for k in range(8):
        sh = 1 << k
        mrot = jnp.roll(m, -sh, axis=1)
        m = jnp.where(((r >> k) & 1) == 0, mrot, m)
    logits = logits + m[:, :n]

    c = n // 8
    ch = [logits[:, i * c:(i + 1) * c] for i in range(8)]
    for i, j in _SORT8:
        hi = jnp.maximum(ch[i], ch[j])
        lo = jnp.minimum(ch[i], ch[j])
        ch[i], ch[j] = hi, lo

    rowmax = jnp.max(ch[0], axis=1, keepdims=True)
    neg = jnp.float32(-jnp.inf)

    def body(_, carry):
        cs = carry[:8]
        m1 = jnp.max(cs[0], axis=1, keepdims=True)
        h1 = cs[0] == m1
        f = jnp.where(h1, cs[1], cs[0])
        m2 = jnp.max(f, axis=1, keepdims=True)
        h2 = f == m2
        e1 = h1 | h2
        e2 = h1 & h2
        ext = cs + (neg, neg)
        new = tuple(
            jnp.where(e2, ext[j + 2], jnp.where(e1, ext[j + 1], cs[j]))
            for j in range(8))
        return new + (m2,)

    out = jax.lax.fori_loop(0, _TOPK // 2, body, tuple(ch) + (rowmax,))
    thresh = out[8]

    p = jnp.where(logits >= thresh, jnp.exp(logits - rowmax), 0.0)
    denom = jnp.sum(p, axis=1, keepdims=True)
    pv = jax.lax.dot_general(
        p.astype(jnp.bfloat16), v_ref[0], (((1,), (0,)), ((), ())),
        preferred_element_type=jnp.float32)
    o_ref[0] = pv / denom


def _ffn_kernel(x_ref, ao_ref, wo_ref, bo_ref, g2_ref, b2_ref,
                w1_ref, bf1_ref, w2_ref, bf2_ref, o_ref):
    x = x_ref[...]
    proj = jax.lax.dot_general(
        ao_ref[...].astype(jnp.bfloat16), wo_ref[...], (((1,), (0,)), ((), ())),
        preferred_element_type=jnp.float32)
    x1 = x + proj + bo_ref[...]
    xn2 = _ln(x1, g2_ref[...], b2_ref[...])
    h = jax.lax.dot_general(
        xn2.astype(jnp.bfloat16), w1_ref[...], (((1,), (0,)), ((), ())),
        preferred_element_type=jnp.float32) + bf1_ref[...]
    h = 0.5 * h * (1.0 + jax.lax.erf(h * 0.7071067811865476))
    ff = jax.lax.dot_general(
        h.astype(jnp.bfloat16), w2_ref[...], (((1,), (0,)), ((), ())),
        preferred_element_type=jnp.float32) + bf2_ref[...]
    o_ref[...] = x1 + ff


def kernel(x, Wq, bq, Wk, bk, Wv, bv, Wo, bo, g1, beta1, g2, beta2,
           W1, bf1, W2, bf2, rel_emb):
    b, n, d = x.shape
    nb = n // _RB
    x2 = x.reshape(n, d)

    wqkv = jnp.concatenate([Wq.T, Wk.T, Wv.T], axis=1)
    bqkv = jnp.concatenate([bq, bk, bv]).reshape(1, 3 * d)
    y = pl.pallas_call(
        _qkv_kernel,
        grid=(nb,),
        in_specs=[
            pl.BlockSpec((_RB, d), lambda i: (i, 0)),
            pl.BlockSpec((d, 3 * d), lambda i: (0, 0)),
            pl.BlockSpec((1, 3 * d), lambda i: (0, 0)),
            pl.BlockSpec((1, d), lambda i: (0, 0)),
            pl.BlockSpec((1, d), lambda i: (0, 0)),
        ],
        out_specs=pl.BlockSpec((_RB, 3 * d), lambda i: (i, 0)),
        out_shape=jax.ShapeDtypeStruct((n, 3 * d), jnp.float32),
    )(x2, wqkv, bqkv, g1.reshape(1, d), beta1.reshape(1, d))

    q, kk, v = jnp.split(y, 3, axis=1)
    qh = q.reshape(n, _H, _HD).transpose(1, 0, 2)
    kth = kk.reshape(n, _H, _HD).transpose(1, 2, 0)
    vh = v.reshape(n, _H, _HD).transpose(1, 0, 2).astype(jnp.bfloat16)

    maxseq = (rel_emb.shape[0] + 1) // 2
    relt = jnp.pad(rel_emb.T, ((0, 0), (0, 1)))
    starts = [maxseq - _RB - bi * _RB for bi in range(nb)]
    relw = jnp.stack(
        [relt[:, s:s + _BIASW] for s in starts], axis=1)
    relw = relw.reshape(_H * nb, 1, _BIASW)

    ao = pl.pallas_call(
        _attn_kernel,
        grid=(_H, nb),
        in_specs=[
            pl.BlockSpec((1, _RB, _HD), lambda h, bi: (h, bi, 0)),
            pl.BlockSpec((1, _HD, n), lambda h, bi: (h, 0, 0)),
            pl.BlockSpec((1, n, _HD), lambda h, bi: (h, 0, 0)),
            pl.BlockSpec((1, 1, _BIASW), lambda h, bi, nb=nb: (h * nb + bi, 0, 0)),
        ],
        out_specs=pl.BlockSpec((1, _RB, _HD), lambda h, bi: (h, bi, 0)),
        out_shape=jax.ShapeDtypeStruct((_H, n, _HD), jnp.float32),
    )(qh, kth, vh, relw)
    ao2 = ao.transpose(1, 0, 2).reshape(n, d)

    out = pl.pallas_call(
        _ffn_kernel,
        grid=(nb,),
        in_specs=[
            pl.BlockSpec((_RB, d), lambda i: (i, 0)),
            pl.BlockSpec((_RB, d), lambda i: (i, 0)),
            pl.BlockSpec((d, d), lambda i: (0, 0)),
            pl.BlockSpec((1, d), lambda i: (0, 0)),
            pl.BlockSpec((1, d), lambda i: (0, 0)),
            pl.BlockSpec((1, d), lambda i: (0, 0)),
            pl.BlockSpec((d, 4 * d), lambda i: (0, 0)),
            pl.BlockSpec((1, 4 * d), lambda i: (0, 0)),
            pl.BlockSpec((4 * d, d), lambda i: (0, 0)),
            pl.BlockSpec((1, d), lambda i: (0, 0)),
        ],
        out_specs=pl.BlockSpec((_RB, d), lambda i: (i, 0)),
        out_shape=jax.ShapeDtypeStruct((n, d), jnp.float32),
    )(x2, ao2, Wo.T.astype(jnp.bfloat16), bo.reshape(1, d),
      g2.reshape(1, d), beta2.reshape(1, d),
      W1.T.astype(jnp.bfloat16), bf1.reshape(1, 4 * d),
      W2.T.astype(jnp.bfloat16), bf2.reshape(1, d))

    return out.reshape(b, n, d)


# speedup vs baseline: 1.5480x; 1.1560x over previous
import jax
import jax.numpy as jnp
from jax.experimental import pallas as pl
from jax.experimental.pallas import tpu as pltpu

_H = 16
_HD = 64
_TOPK = 32
_RB = 256
_BIASW = 2304

_HIGH = jax.lax.Precision.HIGHEST

_SORT8 = [(0, 1), (2, 3), (4, 5), (6, 7),
          (0, 2), (1, 3), (4, 6), (5, 7),
          (1, 2), (5, 6),
          (0, 4), (1, 5), (2, 6), (3, 7),
          (2, 4), (3, 5),
          (1, 2), (3, 4), (5, 6)]


def _ln(x, g, b, eps=1e-5):
    mu = jnp.mean(x, axis=1, keepdims=True)
    xc = x - mu
    var = jnp.mean(xc * xc, axis=1, keepdims=True)
    return xc * jax.lax.rsqrt(var + eps) * g + b


def _qkv_kernel(x_ref, w_ref, b_ref, g_ref, beta_ref, o_ref):
    xn = _ln(x_ref[...], g_ref[...], beta_ref[...])
    o_ref[...] = jax.lax.dot_general(
        xn, w_ref[...], (((1,), (0,)), ((), ())),
        precision=_HIGH, preferred_element_type=jnp.float32) + b_ref[...]


def _attn_kernel(q_ref, kt_ref, v_ref, relw_ref, o_ref):
    n = kt_ref.shape[2]
    scale = _HD ** -0.5
    q = q_ref[0]
    kt = kt_ref[0]
    logits = jax.lax.dot_general(
        q, kt, (((1,), (0,)), ((), ())),
        precision=_HIGH, preferred_element_type=jnp.float32) * scale

    m = jnp.broadcast_to(relw_ref[0], (_RB, _BIASW))
    r = jax.lax.broadcasted_iota(jnp.int32, (_RB, _BIASW), 0)
    for k in range(8):
        sh = 1 << k
        mrot = jnp.roll(m, -sh, axis=1)
        m = jnp.where(((r >> k) & 1) == 0, mrot, m)
    logits = logits + m[:, :n]

    c = n // 8
    ch = [logits[:, i * c:(i + 1) * c] for i in range(8)]
    for i, j in _SORT8:
        hi = jnp.maximum(ch[i], ch[j])
        lo = jnp.minimum(ch[i], ch[j])
        ch[i], ch[j] = hi, lo

    rowmax = jnp.max(ch[0], axis=1, keepdims=True)
    neg = jnp.float32(-jnp.inf)

    def body(_, carry):
        cs = carry[:8]
        f = cs[0]
        cnt = jnp.zeros_like(f)
        m = None
        for k in range(4):
            m = jnp.max(f, axis=1, keepdims=True)
            h = f == m
            cnt = cnt + jnp.where(h, 1.0, 0.0)
            if k == 0:
                f = jnp.where(h, cs[1], f)
            elif k < 3:
                refill = cs[1]
                for t in range(2, k + 2):
                    refill = jnp.where(cnt >= t, cs[t], refill)
                f = jnp.where(h, refill, f)
        ext = cs + (neg, neg, neg, neg)
        new = []
        for j in range(8):
            x = cs[j]
            for t in range(1, 5):
                x = jnp.where(cnt >= t, ext[j + t], x)
            new.append(x)
        return tuple(new) + (m,)

    out = jax.lax.fori_loop(0, _TOPK // 4, body, tuple(ch) + (rowmax,))
    thresh = out[8]

    p = jnp.where(logits >= thresh, jnp.exp(logits - rowmax), 0.0)
    denom = jnp.sum(p, axis=1, keepdims=True)
    pv = jax.lax.dot_general(
        p.astype(jnp.bfloat16), v_ref[0], (((1,), (0,)), ((), ())),
        preferred_element_type=jnp.float32)
    o_ref[0] = pv / denom


def _ffn_kernel(x_ref, ao_ref, wo_ref, bo_ref, g2_ref, b2_ref,
                w1_ref, bf1_ref, w2_ref, bf2_ref, o_ref):
    x = x_ref[...]
    proj = jax.lax.dot_general(
        ao_ref[...].astype(jnp.bfloat16), wo_ref[...], (((1,), (0,)), ((), ())),
        preferred_element_type=jnp.float32)
    x1 = x + proj + bo_ref[...]
    xn2 = _ln(x1, g2_ref[...], b2_ref[...])
    h = jax.lax.dot_general(
        xn2.astype(jnp.bfloat16), w1_ref[...], (((1,), (0,)), ((), ())),
        preferred_element_type=jnp.float32) + bf1_ref[...]
    h = 0.5 * h * (1.0 + jax.lax.erf(h * 0.7071067811865476))
    ff = jax.lax.dot_general(
        h.astype(jnp.bfloat16), w2_ref[...], (((1,), (0,)), ((), ())),
        preferred_element_type=jnp.float32) + bf2_ref[...]
    o_ref[...] = x1 + ff


def kernel(x, Wq, bq, Wk, bk, Wv, bv, Wo, bo, g1, beta1, g2, beta2,
           W1, bf1, W2, bf2, rel_emb):
    b, n, d = x.shape
    nb = n // _RB
    x2 = x.reshape(n, d)

    wqkv = jnp.concatenate([Wq.T, Wk.T, Wv.T], axis=1)
    bqkv = jnp.concatenate([bq, bk, bv]).reshape(1, 3 * d)
    y = pl.pallas_call(
        _qkv_kernel,
        grid=(nb,),
        in_specs=[
            pl.BlockSpec((_RB, d), lambda i: (i, 0)),
            pl.BlockSpec((d, 3 * d), lambda i: (0, 0)),
            pl.BlockSpec((1, 3 * d), lambda i: (0, 0)),
            pl.BlockSpec((1, d), lambda i: (0, 0)),
            pl.BlockSpec((1, d), lambda i: (0, 0)),
        ],
        out_specs=pl.BlockSpec((_RB, 3 * d), lambda i: (i, 0)),
        out_shape=jax.ShapeDtypeStruct((n, 3 * d), jnp.float32),
    )(x2, wqkv, bqkv, g1.reshape(1, d), beta1.reshape(1, d))

    q, kk, v = jnp.split(y, 3, axis=1)
    qh = q.reshape(n, _H, _HD).transpose(1, 0, 2)
    kth = kk.reshape(n, _H, _HD).transpose(1, 2, 0)
    vh = v.reshape(n, _H, _HD).transpose(1, 0, 2).astype(jnp.bfloat16)

    maxseq = (rel_emb.shape[0] + 1) // 2
    relt = jnp.pad(rel_emb.T, ((0, 0), (0, 1)))
    starts = [maxseq - _RB - bi * _RB for bi in range(nb)]
    relw = jnp.stack(
        [relt[:, s:s + _BIASW] for s in starts], axis=1)
    relw = relw.reshape(_H * nb, 1, _BIASW)

    ao = pl.pallas_call(
        _attn_kernel,
        grid=(_H, nb),
        in_specs=[
            pl.BlockSpec((1, _RB, _HD), lambda h, bi: (h, bi, 0)),
            pl.BlockSpec((1, _HD, n), lambda h, bi: (h, 0, 0)),
            pl.BlockSpec((1, n, _HD), lambda h, bi: (h, 0, 0)),
            pl.BlockSpec((1, 1, _BIASW), lambda h, bi, nb=nb: (h * nb + bi, 0, 0)),
        ],
        out_specs=pl.BlockSpec((1, _RB, _HD), lambda h, bi: (h, bi, 0)),
        out_shape=jax.ShapeDtypeStruct((_H, n, _HD), jnp.float32),
    )(qh, kth, vh, relw)
    ao2 = ao.transpose(1, 0, 2).reshape(n, d)

    out = pl.pallas_call(
        _ffn_kernel,
        grid=(nb,),
        in_specs=[
            pl.BlockSpec((_RB, d), lambda i: (i, 0)),
            pl.BlockSpec((_RB, d), lambda i: (i, 0)),
            pl.BlockSpec((d, d), lambda i: (0, 0)),
            pl.BlockSpec((1, d), lambda i: (0, 0)),
            pl.BlockSpec((1, d), lambda i: (0, 0)),
            pl.BlockSpec((1, d), lambda i: (0, 0)),
            pl.BlockSpec((d, 4 * d), lambda i: (0, 0)),
            pl.BlockSpec((1, 4 * d), lambda i: (0, 0)),
            pl.BlockSpec((4 * d, d), lambda i: (0, 0)),
            pl.BlockSpec((1, d), lambda i: (0, 0)),
        ],
        out_specs=pl.BlockSpec((_RB, d), lambda i: (i, 0)),
        out_shape=jax.ShapeDtypeStruct((n, d), jnp.float32),
    )(x2, ao2, Wo.T.astype(jnp.bfloat16), bo.reshape(1, d),
      g2.reshape(1, d), beta2.reshape(1, d),
      W1.T.astype(jnp.bfloat16), bf1.reshape(1, 4 * d),
      W2.T.astype(jnp.bfloat16), bf2.reshape(1, d))

    return out.reshape(b, n, d)


# speedup vs baseline: 1.7283x; 1.1165x over previous
import jax
import jax.numpy as jnp
from jax.experimental import pallas as pl
from jax.experimental.pallas import tpu as pltpu

_H = 16
_HD = 64
_TOPK = 32
_RB = 256
_BIASW = 2304

_HIGH = jax.lax.Precision.HIGHEST

_SORT8 = [(0, 1), (2, 3), (4, 5), (6, 7),
          (0, 2), (1, 3), (4, 6), (5, 7),
          (1, 2), (5, 6),
          (0, 4), (1, 5), (2, 6), (3, 7),
          (2, 4), (3, 5),
          (1, 2), (3, 4), (5, 6)]


def _ln(x, g, b, eps=1e-5):
    mu = jnp.mean(x, axis=1, keepdims=True)
    xc = x - mu
    var = jnp.mean(xc * xc, axis=1, keepdims=True)
    return xc * jax.lax.rsqrt(var + eps) * g + b


def _qkv_kernel(x_ref, w_ref, b_ref, g_ref, beta_ref, o_ref):
    xn = _ln(x_ref[...], g_ref[...], beta_ref[...])
    o_ref[...] = jax.lax.dot_general(
        xn, w_ref[...], (((1,), (0,)), ((), ())),
        precision=_HIGH, preferred_element_type=jnp.float32) + b_ref[...]


def _attn_kernel(q_ref, kt_ref, v_ref, relw_ref, o_ref):
    n = kt_ref.shape[2]
    scale = _HD ** -0.5
    q = q_ref[0]
    kt = kt_ref[0]
    logits = jax.lax.dot_general(
        q, kt, (((1,), (0,)), ((), ())),
        precision=_HIGH, preferred_element_type=jnp.float32) * scale

    m = jnp.broadcast_to(relw_ref[0], (_RB, _BIASW))
    r = jax.lax.broadcasted_iota(jnp.int32, (_RB, _BIASW), 0)
    for k in range(8):
        sh = 1 << k
        mrot = jnp.roll(m, -sh, axis=1)
        m = jnp.where(((r >> k) & 1) == 0, mrot, m)
    logits = logits + m[:, :n]

    c = n // 8
    ch = [logits[:, i * c:(i + 1) * c] for i in range(8)]
    for i, j in _SORT8:
        hi = jnp.maximum(ch[i], ch[j])
        lo = jnp.minimum(ch[i], ch[j])
        ch[i], ch[j] = hi, lo

    rowmax = jnp.max(ch[0], axis=1, keepdims=True)
    neg = jnp.float32(-jnp.inf)

    P = 8

    def body(_, carry):
        cs = carry[:8]
        f = cs[0]
        cnt = jnp.zeros_like(f)
        m = None
        for k in range(P):
            m = jnp.max(f, axis=1, keepdims=True)
            h = f == m
            cnt = cnt + jnp.where(h, 1.0, 0.0)
            if k == 0:
                f = jnp.where(h, cs[1], f)
            elif k < P - 1:
                refill = cs[1]
                for t in range(2, min(k + 1, 7) + 1):
                    refill = jnp.where(cnt >= t, cs[t], refill)
                if k + 1 > 7:
                    refill = jnp.where(cnt >= 8, neg, refill)
                f = jnp.where(h, refill, f)
        ext = cs + (neg,) * P
        new = []
        for j in range(8):
            x = cs[j]
            for t in range(1, P + 1):
                if j + t < 8:
                    x = jnp.where(cnt >= t, ext[j + t], x)
                else:
                    x = jnp.where(cnt >= t, neg, x)
                    break
            new.append(x)
        return tuple(new) + (m,)

    out = jax.lax.fori_loop(0, _TOPK // P, body, tuple(ch) + (rowmax,))
    thresh = out[8]

    p = jnp.where(logits >= thresh, jnp.exp(logits - rowmax), 0.0)
    denom = jnp.sum(p, axis=1, keepdims=True)
    pv = jax.lax.dot_general(
        p.astype(jnp.bfloat16), v_ref[0], (((1,), (0,)), ((), ())),
        preferred_element_type=jnp.float32)
    o_ref[0] = pv / denom


def _ffn_kernel(x_ref, ao_ref, wo_ref, bo_ref, g2_ref, b2_ref,
                w1_ref, bf1_ref, w2_ref, bf2_ref, o_ref):
    x = x_ref[...]
    proj = jax.lax.dot_general(
        ao_ref[...].astype(jnp.bfloat16), wo_ref[...], (((1,), (0,)), ((), ())),
        preferred_element_type=jnp.float32)
    x1 = x + proj + bo_ref[...]
    xn2 = _ln(x1, g2_ref[...], b2_ref[...])
    h = jax.lax.dot_general(
        xn2.astype(jnp.bfloat16), w1_ref[...], (((1,), (0,)), ((), ())),
        preferred_element_type=jnp.float32) + bf1_ref[...]
    h = 0.5 * h * (1.0 + jax.lax.erf(h * 0.7071067811865476))
    ff = jax.lax.dot_general(
        h.astype(jnp.bfloat16), w2_ref[...], (((1,), (0,)), ((), ())),
        preferred_element_type=jnp.float32) + bf2_ref[...]
    o_ref[...] = x1 + ff


def kernel(x, Wq, bq, Wk, bk, Wv, bv, Wo, bo, g1, beta1, g2, beta2,
           W1, bf1, W2, bf2, rel_emb):
    b, n, d = x.shape
    nb = n // _RB
    x2 = x.reshape(n, d)

    wqkv = jnp.concatenate([Wq.T, Wk.T, Wv.T], axis=1)
    bqkv = jnp.concatenate([bq, bk, bv]).reshape(1, 3 * d)
    y = pl.pallas_call(
        _qkv_kernel,
        grid=(nb,),
        in_specs=[
            pl.BlockSpec((_RB, d), lambda i: (i, 0)),
            pl.BlockSpec((d, 3 * d), lambda i: (0, 0)),
            pl.BlockSpec((1, 3 * d), lambda i: (0, 0)),
            pl.BlockSpec((1, d), lambda i: (0, 0)),
            pl.BlockSpec((1, d), lambda i: (0, 0)),
        ],
        out_specs=pl.BlockSpec((_RB, 3 * d), lambda i: (i, 0)),
        out_shape=jax.ShapeDtypeStruct((n, 3 * d), jnp.float32),
    )(x2, wqkv, bqkv, g1.reshape(1, d), beta1.reshape(1, d))

    q, kk, v = jnp.split(y, 3, axis=1)
    qh = q.reshape(n, _H, _HD).transpose(1, 0, 2)
    kth = kk.reshape(n, _H, _HD).transpose(1, 2, 0)
    vh = v.reshape(n, _H, _HD).transpose(1, 0, 2).astype(jnp.bfloat16)

    maxseq = (rel_emb.shape[0] + 1) // 2
    relt = jnp.pad(rel_emb.T, ((0, 0), (0, 1)))
    starts = [maxseq - _RB - bi * _RB for bi in range(nb)]
    relw = jnp.stack(
        [relt[:, s:s + _BIASW] for s in starts], axis=1)
    relw = relw.reshape(_H * nb, 1, _BIASW)

    ao = pl.pallas_call(
        _attn_kernel,
        grid=(_H, nb),
        in_specs=[
            pl.BlockSpec((1, _RB, _HD), lambda h, bi: (h, bi, 0)),
            pl.BlockSpec((1, _HD, n), lambda h, bi: (h, 0, 0)),
            pl.BlockSpec((1, n, _HD), lambda h, bi: (h, 0, 0)),
            pl.BlockSpec((1, 1, _BIASW), lambda h, bi, nb=nb: (h * nb + bi, 0, 0)),
        ],
        out_specs=pl.BlockSpec((1, _RB, _HD), lambda h, bi: (h, bi, 0)),
        out_shape=jax.ShapeDtypeStruct((_H, n, _HD), jnp.float32),
    )(qh, kth, vh, relw)
    ao2 = ao.transpose(1, 0, 2).reshape(n, d)

    out = pl.pallas_call(
        _ffn_kernel,
        grid=(nb,),
        in_specs=[
            pl.BlockSpec((_RB, d), lambda i: (i, 0)),
            pl.BlockSpec((_RB, d), lambda i: (i, 0)),
            pl.BlockSpec((d, d), lambda i: (0, 0)),
            pl.BlockSpec((1, d), lambda i: (0, 0)),
            pl.BlockSpec((1, d), lambda i: (0, 0)),
            pl.BlockSpec((1, d), lambda i: (0, 0)),
            pl.BlockSpec((d, 4 * d), lambda i: (0, 0)),
            pl.BlockSpec((1, 4 * d), lambda i: (0, 0)),
            pl.BlockSpec((4 * d, d), lambda i: (0, 0)),
            pl.BlockSpec((1, d), lambda i: (0, 0)),
        ],
        out_specs=pl.BlockSpec((_RB, d), lambda i: (i, 0)),
        out_shape=jax.ShapeDtypeStruct((n, d), jnp.float32),
    )(x2, ao2, Wo.T.astype(jnp.bfloat16), bo.reshape(1, d),
      g2.reshape(1, d), beta2.reshape(1, d),
      W1.T.astype(jnp.bfloat16), bf1.reshape(1, 4 * d),
      W2.T.astype(jnp.bfloat16), bf2.reshape(1, d))

    return out.reshape(b, n, d)


# speedup vs baseline: 1.7860x; 1.0334x over previous
import jax
import jax.numpy as jnp
from jax.experimental import pallas as pl
from jax.experimental.pallas import tpu as pltpu

_H = 16
_HD = 64
_TOPK = 32
_RB = 256
_BIASW = 2304

_HIGH = jax.lax.Precision.HIGHEST

_SORT8 = [(0, 1), (2, 3), (4, 5), (6, 7),
          (0, 2), (1, 3), (4, 6), (5, 7),
          (1, 2), (5, 6),
          (0, 4), (1, 5), (2, 6), (3, 7),
          (2, 4), (3, 5),
          (1, 2), (3, 4), (5, 6)]


def _ln(x, g, b, eps=1e-5):
    mu = jnp.mean(x, axis=1, keepdims=True)
    xc = x - mu
    var = jnp.mean(xc * xc, axis=1, keepdims=True)
    return xc * jax.lax.rsqrt(var + eps) * g + b


def _qkv_kernel(x_ref, w_ref, b_ref, g_ref, beta_ref, o_ref):
    xn = _ln(x_ref[...], g_ref[...], beta_ref[...])
    o_ref[...] = jax.lax.dot_general(
        xn, w_ref[...], (((1,), (0,)), ((), ())),
        precision=_HIGH, preferred_element_type=jnp.float32) + b_ref[...]


def _attn_kernel(q_ref, kt_ref, v_ref, relw_ref, o_ref):
    n = kt_ref.shape[2]
    scale = _HD ** -0.5
    q = q_ref[0]
    kt = kt_ref[0]
    logits = jax.lax.dot_general(
        q, kt, (((1,), (0,)), ((), ())),
        precision=_HIGH, preferred_element_type=jnp.float32) * scale

    m = jnp.broadcast_to(relw_ref[0], (_RB, _BIASW))
    r = jax.lax.broadcasted_iota(jnp.int32, (_RB, _BIASW), 0)
    for k in range(8):
        sh = 1 << k
        mrot = jnp.roll(m, -sh, axis=1)
        m = jnp.where(((r >> k) & 1) == 0, mrot, m)
    logits = logits + m[:, :n]

    c = n // 8
    ch = [logits[:, i * c:(i + 1) * c] for i in range(8)]
    for i, j in _SORT8:
        hi = jnp.maximum(ch[i], ch[j])
        lo = jnp.minimum(ch[i], ch[j])
        ch[i], ch[j] = hi, lo

    rowmax = jnp.max(ch[0], axis=1, keepdims=True)
    neg = jnp.float32(-jnp.inf)

    P = 16

    def body(_, carry):
        cs = carry[:8]
        f = cs[0]
        cnt = jnp.zeros_like(f)
        m = None
        for k in range(P):
            m = jnp.max(f, axis=1, keepdims=True)
            h = f == m
            cnt = cnt + jnp.where(h, 1.0, 0.0)
            if k == 0:
                f = jnp.where(h, cs[1], f)
            elif k < P - 1:
                refill = cs[1]
                for t in range(2, min(k + 1, 7) + 1):
                    refill = jnp.where(cnt >= t, cs[t], refill)
                if k + 1 > 7:
                    refill = jnp.where(cnt >= 8, neg, refill)
                f = jnp.where(h, refill, f)
        ext = cs + (neg,) * P
        new = []
        for j in range(8):
            x = cs[j]
            for t in range(1, P + 1):
                if j + t < 8:
                    x = jnp.where(cnt >= t, ext[j + t], x)
                else:
                    x = jnp.where(cnt >= t, neg, x)
                    break
            new.append(x)
        return tuple(new) + (m,)

    out = jax.lax.fori_loop(0, _TOPK // P, body, tuple(ch) + (rowmax,))
    thresh = out[8]

    p = jnp.where(logits >= thresh, jnp.exp(logits - rowmax), 0.0)
    denom = jnp.sum(p, axis=1, keepdims=True)
    pv = jax.lax.dot_general(
        p.astype(jnp.bfloat16), v_ref[0], (((1,), (0,)), ((), ())),
        preferred_element_type=jnp.float32)
    o_ref[0] = pv / denom


def _ffn_kernel(x_ref, ao_ref, wo_ref, bo_ref, g2_ref, b2_ref,
                w1_ref, bf1_ref, w2_ref, bf2_ref, o_ref):
    x = x_ref[...]
    proj = jax.lax.dot_general(
        ao_ref[...].astype(jnp.bfloat16), wo_ref[...], (((1,), (0,)), ((), ())),
        preferred_element_type=jnp.float32)
    x1 = x + proj + bo_ref[...]
    xn2 = _ln(x1, g2_ref[...], b2_ref[...])
    h = jax.lax.dot_general(
        xn2.astype(jnp.bfloat16), w1_ref[...], (((1,), (0,)), ((), ())),
        preferred_element_type=jnp.float32) + bf1_ref[...]
    h = 0.5 * h * (1.0 + jax.lax.erf(h * 0.7071067811865476))
    ff = jax.lax.dot_general(
        h.astype(jnp.bfloat16), w2_ref[...], (((1,), (0,)), ((), ())),
        preferred_element_type=jnp.float32) + bf2_ref[...]
    o_ref[...] = x1 + ff


def kernel(x, Wq, bq, Wk, bk, Wv, bv, Wo, bo, g1, beta1, g2, beta2,
           W1, bf1, W2, bf2, rel_emb):
    b, n, d = x.shape
    nb = n // _RB
    x2 = x.reshape(n, d)

    wqkv = jnp.concatenate([Wq.T, Wk.T, Wv.T], axis=1)
    bqkv = jnp.concatenate([bq, bk, bv]).reshape(1, 3 * d)
    y = pl.pallas_call(
        _qkv_kernel,
        grid=(nb,),
        in_specs=[
            pl.BlockSpec((_RB, d), lambda i: (i, 0)),
            pl.BlockSpec((d, 3 * d), lambda i: (0, 0)),
            pl.BlockSpec((1, 3 * d), lambda i: (0, 0)),
            pl.BlockSpec((1, d), lambda i: (0, 0)),
            pl.BlockSpec((1, d), lambda i: (0, 0)),
        ],
        out_specs=pl.BlockSpec((_RB, 3 * d), lambda i: (i, 0)),
        out_shape=jax.ShapeDtypeStruct((n, 3 * d), jnp.float32),
    )(x2, wqkv, bqkv, g1.reshape(1, d), beta1.reshape(1, d))

    q, kk, v = jnp.split(y, 3, axis=1)
    qh = q.reshape(n, _H, _HD).transpose(1, 0, 2)
    kth = kk.reshape(n, _H, _HD).transpose(1, 2, 0)
    vh = v.reshape(n, _H, _HD).transpose(1, 0, 2).astype(jnp.bfloat16)

    maxseq = (rel_emb.shape[0] + 1) // 2
    relt = jnp.pad(rel_emb.T, ((0, 0), (0, 1)))
    starts = [maxseq - _RB - bi * _RB for bi in range(nb)]
    relw = jnp.stack(
        [relt[:, s:s + _BIASW] for s in starts], axis=1)
    relw = relw.reshape(_H * nb, 1, _BIASW)

    ao = pl.pallas_call(
        _attn_kernel,
        grid=(_H, nb),
        in_specs=[
            pl.BlockSpec((1, _RB, _HD), lambda h, bi: (h, bi, 0)),
            pl.BlockSpec((1, _HD, n), lambda h, bi: (h, 0, 0)),
            pl.BlockSpec((1, n, _HD), lambda h, bi: (h, 0, 0)),
            pl.BlockSpec((1, 1, _BIASW), lambda h, bi, nb=nb: (h * nb + bi, 0, 0)),
        ],
        out_specs=pl.BlockSpec((1, _RB, _HD), lambda h, bi: (h, bi, 0)),
        out_shape=jax.ShapeDtypeStruct((_H, n, _HD), jnp.float32),
    )(qh, kth, vh, relw)
    ao2 = ao.transpose(1, 0, 2).reshape(n, d)

    out = pl.pallas_call(
        _ffn_kernel,
        grid=(nb,),
        in_specs=[
            pl.BlockSpec((_RB, d), lambda i: (i, 0)),
            pl.BlockSpec((_RB, d), lambda i: (i, 0)),
            pl.BlockSpec((d, d), lambda i: (0, 0)),
            pl.BlockSpec((1, d), lambda i: (0, 0)),
            pl.BlockSpec((1, d), lambda i: (0, 0)),
            pl.BlockSpec((1, d), lambda i: (0, 0)),
            pl.BlockSpec((d, 4 * d), lambda i: (0, 0)),
            pl.BlockSpec((1, 4 * d), lambda i: (0, 0)),
            pl.BlockSpec((4 * d, d), lambda i: (0, 0)),
            pl.BlockSpec((1, d), lambda i: (0, 0)),
        ],
        out_specs=pl.BlockSpec((_RB, d), lambda i: (i, 0)),
        out_shape=jax.ShapeDtypeStruct((n, d), jnp.float32),
    )(x2, ao2, Wo.T.astype(jnp.bfloat16), bo.reshape(1, d),
      g2.reshape(1, d), beta2.reshape(1, d),
      W1.T.astype(jnp.bfloat16), bf1.reshape(1, 4 * d),
      W2.T.astype(jnp.bfloat16), bf2.reshape(1, d))

    return out.reshape(b, n, d)


# speedup vs baseline: 1.9526x; 1.0933x over previous
import jax
import jax.numpy as jnp
from jax.experimental import pallas as pl
from jax.experimental.pallas import tpu as pltpu

_H = 16
_HD = 64
_TOPK = 32
_RB = 256
_BIASW = 2304

_HIGH = jax.lax.Precision.HIGHEST

_SORT8 = [(0, 1), (2, 3), (4, 5), (6, 7),
          (0, 2), (1, 3), (4, 6), (5, 7),
          (1, 2), (5, 6),
          (0, 4), (1, 5), (2, 6), (3, 7),
          (2, 4), (3, 5),
          (1, 2), (3, 4), (5, 6)]


def _ln(x, g, b, eps=1e-5):
    mu = jnp.mean(x, axis=1, keepdims=True)
    xc = x - mu
    var = jnp.mean(xc * xc, axis=1, keepdims=True)
    return xc * jax.lax.rsqrt(var + eps) * g + b


def _qkv_kernel(x_ref, w_ref, b_ref, g_ref, beta_ref, o_ref):
    xn = _ln(x_ref[...], g_ref[...], beta_ref[...])
    o_ref[...] = jax.lax.dot_general(
        xn, w_ref[...], (((1,), (0,)), ((), ())),
        precision=_HIGH, preferred_element_type=jnp.float32) + b_ref[...]


def _attn_kernel(q_ref, kt_ref, v_ref, relw_ref, o_ref):
    n = kt_ref.shape[2]
    scale = _HD ** -0.5
    q = q_ref[0]
    kt = kt_ref[0]
    logits = jax.lax.dot_general(
        q, kt, (((1,), (0,)), ((), ())),
        precision=_HIGH, preferred_element_type=jnp.float32) * scale

    m = jnp.broadcast_to(relw_ref[0], (_RB, _BIASW))
    r = jax.lax.broadcasted_iota(jnp.int32, (_RB, _BIASW), 0)
    for k in range(8):
        sh = 1 << k
        mrot = jnp.roll(m, -sh, axis=1)
        m = jnp.where(((r >> k) & 1) == 0, mrot, m)
    logits = logits + m[:, :n]

    c = n // 8
    ch = [logits[:, i * c:(i + 1) * c] for i in range(8)]
    for i, j in _SORT8:
        hi = jnp.maximum(ch[i], ch[j])
        lo = jnp.minimum(ch[i], ch[j])
        ch[i], ch[j] = hi, lo

    rowmax = jnp.max(ch[0], axis=1, keepdims=True)
    neg = jnp.float32(-jnp.inf)

    f = ch[0]
    cnt = jnp.zeros_like(f)
    m = None
    for k in range(_TOPK):
        m = jnp.max(f, axis=1, keepdims=True)
        h = f == m
        cnt = cnt + jnp.where(h, 1.0, 0.0)
        if k == _TOPK - 1:
            break
        if k == 0:
            f = jnp.where(h, ch[1], f)
        else:
            refill = ch[1]
            for t in range(2, min(k + 1, 7) + 1):
                refill = jnp.where(cnt >= t, ch[t], refill)
            if k + 1 > 7:
                refill = jnp.where(cnt >= 8, neg, refill)
            f = jnp.where(h, refill, f)
    thresh = m

    p = jnp.where(logits >= thresh, jnp.exp(logits - rowmax), 0.0)
    denom = jnp.sum(p, axis=1, keepdims=True)
    pv = jax.lax.dot_general(
        p.astype(jnp.bfloat16), v_ref[0], (((1,), (0,)), ((), ())),
        preferred_element_type=jnp.float32)
    o_ref[0] = pv / denom


def _ffn_kernel(x_ref, ao_ref, wo_ref, bo_ref, g2_ref, b2_ref,
                w1_ref, bf1_ref, w2_ref, bf2_ref, o_ref):
    x = x_ref[...]
    proj = jax.lax.dot_general(
        ao_ref[...].astype(jnp.bfloat16), wo_ref[...], (((1,), (0,)), ((), ())),
        preferred_element_type=jnp.float32)
    x1 = x + proj + bo_ref[...]
    xn2 = _ln(x1, g2_ref[...], b2_ref[...])
    h = jax.lax.dot_general(
        xn2.astype(jnp.bfloat16), w1_ref[...], (((1,), (0,)), ((), ())),
        preferred_element_type=jnp.float32) + bf1_ref[...]
    h = 0.5 * h * (1.0 + jax.lax.erf(h * 0.7071067811865476))
    ff = jax.lax.dot_general(
        h.astype(jnp.bfloat16), w2_ref[...], (((1,), (0,)), ((), ())),
        preferred_element_type=jnp.float32) + bf2_ref[...]
    o_ref[...] = x1 + ff


def kernel(x, Wq, bq, Wk, bk, Wv, bv, Wo, bo, g1, beta1, g2, beta2,
           W1, bf1, W2, bf2, rel_emb):
    b, n, d = x.shape
    nb = n // _RB
    x2 = x.reshape(n, d)

    wqkv = jnp.concatenate([Wq.T, Wk.T, Wv.T], axis=1)
    bqkv = jnp.concatenate([bq, bk, bv]).reshape(1, 3 * d)
    y = pl.pallas_call(
        _qkv_kernel,
        grid=(nb,),
        in_specs=[
            pl.BlockSpec((_RB, d), lambda i: (i, 0)),
            pl.BlockSpec((d, 3 * d), lambda i: (0, 0)),
            pl.BlockSpec((1, 3 * d), lambda i: (0, 0)),
            pl.BlockSpec((1, d), lambda i: (0, 0)),
            pl.BlockSpec((1, d), lambda i: (0, 0)),
        ],
        out_specs=pl.BlockSpec((_RB, 3 * d), lambda i: (i, 0)),
        out_shape=jax.ShapeDtypeStruct((n, 3 * d), jnp.float32),
    )(x2, wqkv, bqkv, g1.reshape(1, d), beta1.reshape(1, d))

    q, kk, v = jnp.split(y, 3, axis=1)
    qh = q.reshape(n, _H, _HD).transpose(1, 0, 2)
    kth = kk.reshape(n, _H, _HD).transpose(1, 2, 0)
    vh = v.reshape(n, _H, _HD).transpose(1, 0, 2).astype(jnp.bfloat16)

    maxseq = (rel_emb.shape[0] + 1) // 2
    relt = jnp.pad(rel_emb.T, ((0, 0), (0, 1)))
    starts = [maxseq - _RB - bi * _RB for bi in range(nb)]
    relw = jnp.stack(
        [relt[:, s:s + _BIASW] for s in starts], axis=1)
    relw = relw.reshape(_H * nb, 1, _BIASW)

    ao = pl.pallas_call(
        _attn_kernel,
        grid=(_H, nb),
        in_specs=[
            pl.BlockSpec((1, _RB, _HD), lambda h, bi: (h, bi, 0)),
            pl.BlockSpec((1, _HD, n), lambda h, bi: (h, 0, 0)),
            pl.BlockSpec((1, n, _HD), lambda h, bi: (h, 0, 0)),
            pl.BlockSpec((1, 1, _BIASW), lambda h, bi, nb=nb: (h * nb + bi, 0, 0)),
        ],
        out_specs=pl.BlockSpec((1, _RB, _HD), lambda h, bi: (h, bi, 0)),
        out_shape=jax.ShapeDtypeStruct((_H, n, _HD), jnp.float32),
    )(qh, kth, vh, relw)
    ao2 = ao.transpose(1, 0, 2).reshape(n, d)

    out = pl.pallas_call(
        _ffn_kernel,
        grid=(nb,),
        in_specs=[
            pl.BlockSpec((_RB, d), lambda i: (i, 0)),
            pl.BlockSpec((_RB, d), lambda i: (i, 0)),
            pl.BlockSpec((d, d), lambda i: (0, 0)),
            pl.BlockSpec((1, d), lambda i: (0, 0)),
            pl.BlockSpec((1, d), lambda i: (0, 0)),
            pl.BlockSpec((1, d), lambda i: (0, 0)),
            pl.BlockSpec((d, 4 * d), lambda i: (0, 0)),
            pl.BlockSpec((1, 4 * d), lambda i: (0, 0)),
            pl.BlockSpec((4 * d, d), lambda i: (0, 0)),
            pl.BlockSpec((1, d), lambda i: (0, 0)),
        ],
        out_specs=pl.BlockSpec((_RB, d), lambda i: (i, 0)),
        out_shape=jax.ShapeDtypeStruct((n, d), jnp.float32),
    )(x2, ao2, Wo.T.astype(jnp.bfloat16), bo.reshape(1, d),
      g2.reshape(1, d), beta2.reshape(1, d),
      W1.T.astype(jnp.bfloat16), bf1.reshape(1, 4 * d),
      W2.T.astype(jnp.bfloat16), bf2.reshape(1, d))

    return out.reshape(b, n, d)
